# baseline (speedup 1.0000x reference)
import numpy as np

import jax
import jax.numpy as jnp
from jax.experimental import pallas as pl
from jax.experimental.pallas import tpu as pltpu

B, S, HID = 2, 1024, 1024
NH, D = 16, 64
CLIP = 5.0
EPS = 1e-6

CH = 64
NC = S // CH
UNROLL = 4
TL = 256


def _proj_body(x_ref, wq_ref, wk_ref, wv_ref, wab_ref, bab_ref, m_ref,
               seg_ref, segt_ref,
               qc_ref, kc_ref, v_ref, a_ref, b_ref):
    x = x_ref[...]
    xb = x.astype(jnp.bfloat16)
    seg = seg_ref[...]
    segt = segt_ref[...]
    m = m_ref[...]

    pq = jnp.dot(xb, wq_ref[...], preferred_element_type=jnp.float32)
    sq = jnp.dot(pq * pq, seg, preferred_element_type=jnp.float32)
    dq = 1.0 / jnp.maximum(jnp.sqrt(sq), EPS)
    qn = pq * jnp.dot(dq, segt, preferred_element_type=jnp.float32)
    qc_ref[...] = (qn * m).astype(jnp.bfloat16)

    pk = jnp.dot(xb, wk_ref[...], preferred_element_type=jnp.float32)
    sk = jnp.dot(pk * pk, seg, preferred_element_type=jnp.float32)
    dk = 1.0 / jnp.maximum(jnp.sqrt(sk), EPS)
    kc_ref[...] = (pk * jnp.dot(dk, segt,
                                preferred_element_type=jnp.float32)
                   ).astype(jnp.bfloat16)

    ab = jnp.dot(x, wab_ref[...], preferred_element_type=jnp.float32)
    ab = jax.nn.sigmoid(ab + bab_ref[...])
    al = jnp.where(m > 0.0, ab[:, :NH], 1.0)
    be = ab[:, NH:] * m
    a_ref[...] = jnp.dot(al, segt, preferred_element_type=jnp.float32)
    b_ref[...] = jnp.dot(be, segt, preferred_element_type=jnp.float32)

    v_ref[...] = jnp.tanh(
        jnp.dot(xb, wv_ref[...], preferred_element_type=jnp.float32))


def _sumk(t):
    return jnp.sum(t, axis=0, keepdims=True)


def _scan_body(kc_ref, qc_ref, v_ref, a_ref, b_ref, s0_ref, segt_ref,
               y_ref, sf_ref, st_ref):
    c = pl.program_id(0)

    @pl.when(c == 0)
    def _():
        st_ref[...] = s0_ref[...]

    segt = segt_ref[...]

    dn = (((0,), (0,)), ((), ()))

    def step(t, _):
        for b in range(B):
            kcolt = kc_ref[b, t]
            qcolt = qc_ref[b, t]
            vrow = v_ref[b, pl.ds(t, 1), :]
            arow = a_ref[b, pl.ds(t, 1), :]
            brow = b_ref[b, pl.ds(t, 1), :]
            for lt in range(HID // TL):
                sl = slice(lt * TL, (lt + 1) * TL)
                hsl = slice(lt * (TL // D), (lt + 1) * (TL // D))
                kbc = jax.lax.dot_general(
                    kcolt[hsl, :], segt[hsl, sl], dn,
                    preferred_element_type=jnp.float32)
                qbc = jax.lax.dot_general(
                    qcolt[hsl, :], segt[hsl, sl], dn,
                    preferred_element_type=jnp.float32)
                st = st_ref[b, :, sl]
                rd = _sumk(st * kbc)
                dsc = (vrow[:, sl] - rd) * brow[:, sl]
                ns = jax.lax.clamp(-CLIP, arow[:, sl] * st + dsc * kbc,
                                   CLIP)
                st_ref[b, :, sl] = ns
                y_ref[b, pl.ds(t, 1), sl] = _sumk(ns * qbc)
        return ()

    jax.lax.fori_loop(0, CH, step, (), unroll=UNROLL)
    sf_ref[...] = st_ref[...]


def _out_body(y_ref, wo_ref, o_ref):
    o_ref[...] = jnp.dot(y_ref[...], wo_ref[...],
                         preferred_element_type=jnp.float32)


def kernel(x, Wq, Wk, Wv, Wa, ba, Wb, bb, Wo, state0, attention_mask):
    f32 = jnp.float32
    BS = B * S

    cols = np.arange(HID)
    seg = np.zeros((HID, NH), np.float32)
    seg[cols, cols // D] = 1.0
    seg = jnp.asarray(seg)
    segt = seg.T
    segt_bf = segt.astype(jnp.bfloat16)

    xf = x.reshape(BS, HID)
    wab = jnp.concatenate([Wa, Wb], axis=1)
    bab = jnp.concatenate([ba, bb]).reshape(1, 2 * NH)
    mf = attention_mask.astype(f32).reshape(BS, 1)

    RM = 128
    nm = BS // RM
    row_spec = pl.BlockSpec((RM, HID), lambda i: (i, 0))
    full = lambda *shape: pl.BlockSpec(shape, lambda i: (0,) * len(shape))
    qc_a, kc_a, v_a, a_a, b_a = pl.pallas_call(
        _proj_body,
        grid=(nm,),
        in_specs=[row_spec, full(HID, HID), full(HID, HID), full(HID, HID),
                  full(HID, 2 * NH), full(1, 2 * NH),
                  pl.BlockSpec((RM, 1), lambda i: (i, 0)),
                  full(HID, NH), full(NH, HID)],
        out_specs=[row_spec] * 5,
        out_shape=[jax.ShapeDtypeStruct((BS, HID), jnp.bfloat16)] * 2
        + [jax.ShapeDtypeStruct((BS, HID), f32)] * 3,
        compiler_params=pltpu.CompilerParams(
            dimension_semantics=("arbitrary",)),
        name="gdn_proj",
    )(xf, Wq.astype(jnp.bfloat16), Wk.astype(jnp.bfloat16),
      Wv.astype(jnp.bfloat16), wab, bab, mf, seg, segt)

    kc_in = kc_a.reshape(B, S, NH, D)
    qc_in = qc_a.reshape(B, S, NH, D)
    v_in = v_a.reshape(B, S, HID)
    a_in = a_a.reshape(B, S, HID)
    b_in = b_a.reshape(B, S, HID)
    s0 = state0.transpose(0, 3, 1, 2).reshape(B, D, HID)

    t_spec = pl.BlockSpec((B, CH, HID), lambda c: (0, c, 0))
    c_spec = pl.BlockSpec((B, CH, NH, D), lambda c: (0, c, 0, 0))
    s_spec = pl.BlockSpec((B, D, HID), lambda c: (0, 0, 0))
    y_a, sf = pl.pallas_call(
        _scan_body,
        grid=(NC,),
        in_specs=[c_spec, c_spec, t_spec, t_spec, t_spec, s_spec,
                  pl.BlockSpec((NH, HID), lambda c: (0, 0))],
        out_specs=[t_spec, s_spec],
        out_shape=[jax.ShapeDtypeStruct((B, S, HID), f32),
                   jax.ShapeDtypeStruct((B, D, HID), f32)],
        scratch_shapes=[pltpu.VMEM((B, D, HID), f32)],
        compiler_params=pltpu.CompilerParams(
            dimension_semantics=("arbitrary",)),
        name="gdn_scan",
    )(kc_in, qc_in, v_in, a_in, b_in, s0, segt_bf)

    RO = 512
    no = BS // RO
    yf = y_a.reshape(BS, HID)
    of = pl.pallas_call(
        _out_body,
        grid=(no,),
        in_specs=[pl.BlockSpec((RO, HID), lambda i: (i, 0)),
                  pl.BlockSpec((HID, HID), lambda i: (0, 0))],
        out_specs=pl.BlockSpec((RO, HID), lambda i: (i, 0)),
        out_shape=jax.ShapeDtypeStruct((BS, HID), f32),
        compiler_params=pltpu.CompilerParams(
            dimension_semantics=("arbitrary",)),
        name="gdn_out",
    )(yf, Wo)

    out = of.reshape(B, S, HID)
    final_state = sf.reshape(B, D, NH, D).transpose(0, 2, 3, 1)
    return out, final_state


# speedup vs baseline: 1.0116x; 1.0116x over previous
import numpy as np

import jax
import jax.numpy as jnp
from jax.experimental import pallas as pl
from jax.experimental.pallas import tpu as pltpu

B, S, HID = 2, 1024, 1024
NH, D = 16, 64
CLIP = 5.0
EPS = 1e-6

CH = 64
NC = S // CH
UNROLL = 4
TL = 128


def _proj_body(x_ref, wq_ref, wk_ref, wv_ref, wab_ref, bab_ref, m_ref,
               seg_ref, segt_ref,
               qc_ref, kc_ref, v_ref, a_ref, b_ref):
    x = x_ref[...]
    xb = x.astype(jnp.bfloat16)
    seg = seg_ref[...]
    segt = segt_ref[...]
    m = m_ref[...]

    pq = jnp.dot(xb, wq_ref[...], preferred_element_type=jnp.float32)
    sq = jnp.dot(pq * pq, seg, preferred_element_type=jnp.float32)
    dq = 1.0 / jnp.maximum(jnp.sqrt(sq), EPS)
    qn = pq * jnp.dot(dq, segt, preferred_element_type=jnp.float32)
    qc_ref[...] = (qn * m).astype(jnp.bfloat16)

    pk = jnp.dot(xb, wk_ref[...], preferred_element_type=jnp.float32)
    sk = jnp.dot(pk * pk, seg, preferred_element_type=jnp.float32)
    dk = 1.0 / jnp.maximum(jnp.sqrt(sk), EPS)
    kc_ref[...] = (pk * jnp.dot(dk, segt,
                                preferred_element_type=jnp.float32)
                   ).astype(jnp.bfloat16)

    ab = jnp.dot(x, wab_ref[...], preferred_element_type=jnp.float32)
    ab = jax.nn.sigmoid(ab + bab_ref[...])
    al = jnp.where(m > 0.0, ab[:, :NH], 1.0)
    be = ab[:, NH:] * m
    a_ref[...] = jnp.dot(al, segt, preferred_element_type=jnp.float32)
    b_ref[...] = jnp.dot(be, segt, preferred_element_type=jnp.float32)

    v_ref[...] = jnp.tanh(
        jnp.dot(xb, wv_ref[...], preferred_element_type=jnp.float32))


def _sumk(t):
    return jnp.sum(t, axis=0, keepdims=True)


def _scan_body(kc_ref, qc_ref, v_ref, a_ref, b_ref, s0_ref, segt_ref,
               y_ref, sf_ref, st_ref):
    c = pl.program_id(0)

    @pl.when(c == 0)
    def _():
        st_ref[...] = s0_ref[...]

    segt = segt_ref[...]

    dn = (((0,), (0,)), ((), ()))

    def step(t, _):
        for b in range(B):
            kcolt = kc_ref[b, t]
            qcolt = qc_ref[b, t]
            vrow = v_ref[b, pl.ds(t, 1), :]
            arow = a_ref[b, pl.ds(t, 1), :]
            brow = b_ref[b, pl.ds(t, 1), :]
            yparts = []
            for lt in range(HID // TL):
                sl = slice(lt * TL, (lt + 1) * TL)
                hsl = slice(lt * (TL // D), (lt + 1) * (TL // D))
                kbc = jax.lax.dot_general(
                    kcolt[hsl, :], segt[hsl, sl], dn,
                    preferred_element_type=jnp.float32)
                qbc = jax.lax.dot_general(
                    qcolt[hsl, :], segt[hsl, sl], dn,
                    preferred_element_type=jnp.float32)
                st = st_ref[b, :, sl]
                rd = _sumk(st * kbc)
                dsc = (vrow[:, sl] - rd) * brow[:, sl]
                ns = jax.lax.clamp(-CLIP, arow[:, sl] * st + dsc * kbc,
                                   CLIP)
                st_ref[b, :, sl] = ns
                yparts.append(_sumk(ns * qbc))
            y_ref[b, pl.ds(t, 1), :] = jnp.concatenate(yparts, axis=1)
        return ()

    jax.lax.fori_loop(0, CH, step, (), unroll=UNROLL)
    sf_ref[...] = st_ref[...]


def _out_body(y_ref, wo_ref, o_ref):
    o_ref[...] = jnp.dot(y_ref[...], wo_ref[...],
                         preferred_element_type=jnp.float32)


def kernel(x, Wq, Wk, Wv, Wa, ba, Wb, bb, Wo, state0, attention_mask):
    f32 = jnp.float32
    BS = B * S

    cols = np.arange(HID)
    seg = np.zeros((HID, NH), np.float32)
    seg[cols, cols // D] = 1.0
    seg = jnp.asarray(seg)
    segt = seg.T
    segt_bf = segt.astype(jnp.bfloat16)

    xf = x.reshape(BS, HID)
    wab = jnp.concatenate([Wa, Wb], axis=1)
    bab = jnp.concatenate([ba, bb]).reshape(1, 2 * NH)
    mf = attention_mask.astype(f32).reshape(BS, 1)

    RM = 256
    nm = BS // RM
    row_spec = pl.BlockSpec((RM, HID), lambda i: (i, 0))
    full = lambda *shape: pl.BlockSpec(shape, lambda i: (0,) * len(shape))
    qc_a, kc_a, v_a, a_a, b_a = pl.pallas_call(
        _proj_body,
        grid=(nm,),
        in_specs=[row_spec, full(HID, HID), full(HID, HID), full(HID, HID),
                  full(HID, 2 * NH), full(1, 2 * NH),
                  pl.BlockSpec((RM, 1), lambda i: (i, 0)),
                  full(HID, NH), full(NH, HID)],
        out_specs=[row_spec] * 5,
        out_shape=[jax.ShapeDtypeStruct((BS, HID), jnp.bfloat16)] * 2
        + [jax.ShapeDtypeStruct((BS, HID), f32)] * 3,
        compiler_params=pltpu.CompilerParams(
            dimension_semantics=("arbitrary",)),
        name="gdn_proj",
    )(xf, Wq.astype(jnp.bfloat16), Wk.astype(jnp.bfloat16),
      Wv.astype(jnp.bfloat16), wab, bab, mf, seg, segt)

    kc_in = kc_a.reshape(B, S, NH, D)
    qc_in = qc_a.reshape(B, S, NH, D)
    v_in = v_a.reshape(B, S, HID)
    a_in = a_a.reshape(B, S, HID)
    b_in = b_a.reshape(B, S, HID)
    s0 = state0.transpose(0, 3, 1, 2).reshape(B, D, HID)

    t_spec = pl.BlockSpec((B, CH, HID), lambda c: (0, c, 0))
    c_spec = pl.BlockSpec((B, CH, NH, D), lambda c: (0, c, 0, 0))
    s_spec = pl.BlockSpec((B, D, HID), lambda c: (0, 0, 0))
    y_a, sf = pl.pallas_call(
        _scan_body,
        grid=(NC,),
        in_specs=[c_spec, c_spec, t_spec, t_spec, t_spec, s_spec,
                  pl.BlockSpec((NH, HID), lambda c: (0, 0))],
        out_specs=[t_spec, s_spec],
        out_shape=[jax.ShapeDtypeStruct((B, S, HID), f32),
                   jax.ShapeDtypeStruct((B, D, HID), f32)],
        scratch_shapes=[pltpu.VMEM((B, D, HID), f32)],
        compiler_params=pltpu.CompilerParams(
            dimension_semantics=("arbitrary",)),
        name="gdn_scan",
    )(kc_in, qc_in, v_in, a_in, b_in, s0, segt_bf)

    RO = 512
    no = BS // RO
    yf = y_a.reshape(BS, HID).astype(jnp.bfloat16)
    of = pl.pallas_call(
        _out_body,
        grid=(no,),
        in_specs=[pl.BlockSpec((RO, HID), lambda i: (i, 0)),
                  pl.BlockSpec((HID, HID), lambda i: (0, 0))],
        out_specs=pl.BlockSpec((RO, HID), lambda i: (i, 0)),
        out_shape=jax.ShapeDtypeStruct((BS, HID), f32),
        compiler_params=pltpu.CompilerParams(
            dimension_semantics=("arbitrary",)),
        name="gdn_out",
    )(yf, Wo.astype(jnp.bfloat16))

    out = of.reshape(B, S, HID)
    final_state = sf.reshape(B, D, NH, D).transpose(0, 2, 3, 1)
    return out, final_state


# speedup vs baseline: 1.0299x; 1.0181x over previous
import numpy as np

import jax
import jax.numpy as jnp
from jax.experimental import pallas as pl
from jax.experimental.pallas import tpu as pltpu

B, S, HID = 2, 1024, 1024
NH, D = 16, 64
CLIP = 5.0
EPS = 1e-6

CH = 128
NC = S // CH
UNROLL = 4
TL = 128


def _proj_body(x_ref, wq_ref, wk_ref, wv_ref, wab_ref, bab_ref, m_ref,
               seg_ref, segt_ref,
               qc_ref, kc_ref, v_ref, a_ref, b_ref):
    x = x_ref[...]
    seg = seg_ref[...]
    segt = segt_ref[...]
    m = m_ref[...]

    pq = jnp.dot(x, wq_ref[...], preferred_element_type=jnp.float32)
    sq = jnp.dot(pq * pq, seg, preferred_element_type=jnp.float32)
    dq = 1.0 / jnp.maximum(jnp.sqrt(sq), EPS)
    qn = pq * jnp.dot(dq, segt, preferred_element_type=jnp.float32)
    qc_ref[...] = (qn * m).astype(jnp.bfloat16)

    pk = jnp.dot(x, wk_ref[...], preferred_element_type=jnp.float32)
    sk = jnp.dot(pk * pk, seg, preferred_element_type=jnp.float32)
    dk = 1.0 / jnp.maximum(jnp.sqrt(sk), EPS)
    kc_ref[...] = (pk * jnp.dot(dk, segt,
                                preferred_element_type=jnp.float32)
                   ).astype(jnp.bfloat16)

    ab = jnp.dot(x, wab_ref[...], preferred_element_type=jnp.float32)
    ab = jax.nn.sigmoid(ab + bab_ref[...])
    al = jnp.where(m > 0.0, ab[:, :NH], 1.0)
    be = ab[:, NH:] * m
    a_ref[...] = jnp.dot(al, segt, preferred_element_type=jnp.float32)
    b_ref[...] = jnp.dot(be, segt, preferred_element_type=jnp.float32)

    v_ref[...] = jnp.tanh(
        jnp.dot(x, wv_ref[...], preferred_element_type=jnp.float32))


def _sumk(t):
    return jnp.sum(t, axis=0, keepdims=True)


def _scan_body(kc_ref, qc_ref, v_ref, a_ref, b_ref, s0_ref, segt_ref,
               y_ref, sf_ref, st_ref):
    c = pl.program_id(0)

    @pl.when(c == 0)
    def _():
        st_ref[...] = s0_ref[...]

    segt = segt_ref[...]

    dn = (((0,), (0,)), ((), ()))

    def step(t, _):
        for b in range(B):
            kcolt = kc_ref[b, t]
            qcolt = qc_ref[b, t]
            vrow = v_ref[b, pl.ds(t, 1), :]
            arow = a_ref[b, pl.ds(t, 1), :]
            brow = b_ref[b, pl.ds(t, 1), :]
            yparts = []
            for lt in range(HID // TL):
                sl = slice(lt * TL, (lt + 1) * TL)
                hsl = slice(lt * (TL // D), (lt + 1) * (TL // D))
                kbc = jax.lax.dot_general(
                    kcolt[hsl, :], segt[hsl, sl], dn,
                    preferred_element_type=jnp.float32)
                qbc = jax.lax.dot_general(
                    qcolt[hsl, :], segt[hsl, sl], dn,
                    preferred_element_type=jnp.float32)
                st = st_ref[b, :, sl]
                rd = _sumk(st * kbc)
                dsc = (vrow[:, sl] - rd) * brow[:, sl]
                ns = jax.lax.clamp(-CLIP, arow[:, sl] * st + dsc * kbc,
                                   CLIP)
                st_ref[b, :, sl] = ns
                yparts.append(_sumk(ns * qbc))
            y_ref[b, pl.ds(t, 1), :] = jnp.concatenate(yparts, axis=1)
        return ()

    jax.lax.fori_loop(0, CH, step, (), unroll=UNROLL)
    sf_ref[...] = st_ref[...]


def _out_body(y_ref, wo_ref, o_ref):
    o_ref[...] = jnp.dot(y_ref[...], wo_ref[...],
                         preferred_element_type=jnp.float32)


def kernel(x, Wq, Wk, Wv, Wa, ba, Wb, bb, Wo, state0, attention_mask):
    f32 = jnp.float32
    BS = B * S

    cols = np.arange(HID)
    seg = np.zeros((HID, NH), np.float32)
    seg[cols, cols // D] = 1.0
    seg = jnp.asarray(seg)
    segt = seg.T
    segt_bf = segt.astype(jnp.bfloat16)

    xf = x.reshape(BS, HID)
    wab = jnp.concatenate([Wa, Wb], axis=1)
    bab = jnp.concatenate([ba, bb]).reshape(1, 2 * NH)
    mf = attention_mask.astype(f32).reshape(BS, 1)

    RM = 256
    nm = BS // RM
    row_spec = pl.BlockSpec((RM, HID), lambda i: (i, 0))
    full = lambda *shape: pl.BlockSpec(shape, lambda i: (0,) * len(shape))
    qc_a, kc_a, v_a, a_a, b_a = pl.pallas_call(
        _proj_body,
        grid=(nm,),
        in_specs=[row_spec, full(HID, HID), full(HID, HID), full(HID, HID),
                  full(HID, 2 * NH), full(1, 2 * NH),
                  pl.BlockSpec((RM, 1), lambda i: (i, 0)),
                  full(HID, NH), full(NH, HID)],
        out_specs=[row_spec] * 5,
        out_shape=[jax.ShapeDtypeStruct((BS, HID), jnp.bfloat16)] * 2
        + [jax.ShapeDtypeStruct((BS, HID), f32)] * 3,
        compiler_params=pltpu.CompilerParams(
            dimension_semantics=("arbitrary",)),
        name="gdn_proj",
    )(xf, Wq, Wk, Wv, wab, bab, mf, seg, segt)

    kc_in = kc_a.reshape(B, S, NH, D)
    qc_in = qc_a.reshape(B, S, NH, D)
    v_in = v_a.reshape(B, S, HID)
    a_in = a_a.reshape(B, S, HID)
    b_in = b_a.reshape(B, S, HID)
    s0 = state0.transpose(0, 3, 1, 2).reshape(B, D, HID)

    t_spec = pl.BlockSpec((B, CH, HID), lambda c: (0, c, 0))
    c_spec = pl.BlockSpec((B, CH, NH, D), lambda c: (0, c, 0, 0))
    s_spec = pl.BlockSpec((B, D, HID), lambda c: (0, 0, 0))
    y_a, sf = pl.pallas_call(
        _scan_body,
        grid=(NC,),
        in_specs=[c_spec, c_spec, t_spec, t_spec, t_spec, s_spec,
                  pl.BlockSpec((NH, HID), lambda c: (0, 0))],
        out_specs=[t_spec, s_spec],
        out_shape=[jax.ShapeDtypeStruct((B, S, HID), f32),
                   jax.ShapeDtypeStruct((B, D, HID), f32)],
        scratch_shapes=[pltpu.VMEM((B, D, HID), f32)],
        compiler_params=pltpu.CompilerParams(
            dimension_semantics=("arbitrary",)),
        name="gdn_scan",
    )(kc_in, qc_in, v_in, a_in, b_in, s0, segt_bf)

    RO = 512
    no = BS // RO
    yf = y_a.reshape(BS, HID).astype(jnp.bfloat16)
    of = pl.pallas_call(
        _out_body,
        grid=(no,),
        in_specs=[pl.BlockSpec((RO, HID), lambda i: (i, 0)),
                  pl.BlockSpec((HID, HID), lambda i: (0, 0))],
        out_specs=pl.BlockSpec((RO, HID), lambda i: (i, 0)),
        out_shape=jax.ShapeDtypeStruct((BS, HID), f32),
        compiler_params=pltpu.CompilerParams(
            dimension_semantics=("arbitrary",)),
        name="gdn_out",
    )(yf, Wo.astype(jnp.bfloat16))

    out = of.reshape(B, S, HID)
    final_state = sf.reshape(B, D, NH, D).transpose(0, 2, 3, 1)
    return out, final_state


# speedup vs baseline: 1.1255x; 1.0928x over previous
import numpy as np

import jax
import jax.numpy as jnp
from jax.experimental import pallas as pl
from jax.experimental.pallas import tpu as pltpu

B, S, HID = 2, 1024, 1024
NH, D = 16, 64
CLIP = 5.0
EPS = 1e-6

CH = 128
NC = S // CH
UNROLL = 8
TL = 128


def _proj_body(x_ref, wq_ref, wk_ref, wv_ref, wab_ref, bab_ref, m_ref,
               seg_ref, segt_ref,
               qc_ref, kc_ref, v_ref, a_ref, b_ref):
    x = x_ref[...]
    seg = seg_ref[...]
    segt = segt_ref[...]
    m = m_ref[...]

    pq = jnp.dot(x, wq_ref[...], preferred_element_type=jnp.float32)
    sq = jnp.dot(pq * pq, seg, preferred_element_type=jnp.float32)
    dq = 1.0 / jnp.maximum(jnp.sqrt(sq), EPS)
    qn = pq * jnp.dot(dq, segt, preferred_element_type=jnp.float32)
    qc_ref[...] = (qn * m).astype(jnp.bfloat16)

    pk = jnp.dot(x, wk_ref[...], preferred_element_type=jnp.float32)
    sk = jnp.dot(pk * pk, seg, preferred_element_type=jnp.float32)
    dk = 1.0 / jnp.maximum(jnp.sqrt(sk), EPS)
    kc_ref[...] = (pk * jnp.dot(dk, segt,
                                preferred_element_type=jnp.float32)
                   ).astype(jnp.bfloat16)

    ab = jnp.dot(x, wab_ref[...], preferred_element_type=jnp.float32)
    ab = jax.nn.sigmoid(ab + bab_ref[...])
    al = jnp.where(m > 0.0, ab[:, :NH], 1.0)
    be = ab[:, NH:] * m
    a_ref[...] = jnp.dot(al, segt, preferred_element_type=jnp.float32)
    b_ref[...] = jnp.dot(be, segt, preferred_element_type=jnp.float32)

    v_ref[...] = jnp.tanh(
        jnp.dot(x, wv_ref[...], preferred_element_type=jnp.float32))


def _sumk(t):
    return jnp.sum(t, axis=0, keepdims=True)


def _scan_body(kc_ref, qc_ref, v_ref, a_ref, b_ref, s0_ref, segt_ref,
               y_ref, sf_ref, st_ref):
    c = pl.program_id(0)

    @pl.when(c == 0)
    def _():
        st_ref[...] = s0_ref[...]

    segt = segt_ref[...]

    dn = (((0,), (0,)), ((), ()))

    def step(t, _):
        for b in range(B):
            kcolt = kc_ref[b, t]
            qcolt = qc_ref[b, t]
            vrow = v_ref[b, pl.ds(t, 1), :]
            arow = a_ref[b, pl.ds(t, 1), :]
            brow = b_ref[b, pl.ds(t, 1), :]
            yparts = []
            for lt in range(HID // TL):
                sl = slice(lt * TL, (lt + 1) * TL)
                hsl = slice(lt * (TL // D), (lt + 1) * (TL // D))
                kbc = jax.lax.dot_general(
                    kcolt[hsl, :], segt[hsl, sl], dn,
                    preferred_element_type=jnp.float32)
                qbc = jax.lax.dot_general(
                    qcolt[hsl, :], segt[hsl, sl], dn,
                    preferred_element_type=jnp.float32)
                st = st_ref[b, :, sl]
                rd = _sumk(st * kbc)
                dsc = (vrow[:, sl] - rd) * brow[:, sl]
                ns = jax.lax.clamp(-CLIP, arow[:, sl] * st + dsc * kbc,
                                   CLIP)
                st_ref[b, :, sl] = ns
                yparts.append(_sumk(ns * qbc))
            y_ref[b, pl.ds(t, 1), :] = jnp.concatenate(yparts, axis=1)
        return ()

    jax.lax.fori_loop(0, CH, step, (), unroll=UNROLL)
    sf_ref[...] = st_ref[...]


def _out_body(y_ref, wo_ref, o_ref):
    o_ref[...] = jnp.dot(y_ref[...], wo_ref[...],
                         preferred_element_type=jnp.float32)


def kernel(x, Wq, Wk, Wv, Wa, ba, Wb, bb, Wo, state0, attention_mask):
    f32 = jnp.float32
    BS = B * S

    cols = np.arange(HID)
    seg = np.zeros((HID, NH), np.float32)
    seg[cols, cols // D] = 1.0
    seg = jnp.asarray(seg)
    segt = seg.T
    segt_bf = segt.astype(jnp.bfloat16)

    xf = x.reshape(BS, HID)
    wab = jnp.concatenate([Wa, Wb], axis=1)
    bab = jnp.concatenate([ba, bb]).reshape(1, 2 * NH)
    mf = attention_mask.astype(f32).reshape(BS, 1)

    RM = 256
    nm = BS // RM
    row_spec = pl.BlockSpec((RM, HID), lambda i: (i, 0))
    full = lambda *shape: pl.BlockSpec(shape, lambda i: (0,) * len(shape))
    qc_a, kc_a, v_a, a_a, b_a = pl.pallas_call(
        _proj_body,
        grid=(nm,),
        in_specs=[row_spec, full(HID, HID), full(HID, HID), full(HID, HID),
                  full(HID, 2 * NH), full(1, 2 * NH),
                  pl.BlockSpec((RM, 1), lambda i: (i, 0)),
                  full(HID, NH), full(NH, HID)],
        out_specs=[row_spec] * 5,
        out_shape=[jax.ShapeDtypeStruct((BS, HID), jnp.bfloat16)] * 2
        + [jax.ShapeDtypeStruct((BS, HID), f32)] * 3,
        compiler_params=pltpu.CompilerParams(
            dimension_semantics=("arbitrary",)),
        name="gdn_proj",
    )(xf, Wq, Wk, Wv, wab, bab, mf, seg, segt)

    kc_in = kc_a.reshape(B, S, NH, D)
    qc_in = qc_a.reshape(B, S, NH, D)
    v_in = v_a.reshape(B, S, HID)
    a_in = a_a.reshape(B, S, HID)
    b_in = b_a.reshape(B, S, HID)
    s0 = state0.transpose(0, 3, 1, 2).reshape(B, D, HID)

    t_spec = pl.BlockSpec((B, CH, HID), lambda c: (0, c, 0))
    c_spec = pl.BlockSpec((B, CH, NH, D), lambda c: (0, c, 0, 0))
    s_spec = pl.BlockSpec((B, D, HID), lambda c: (0, 0, 0))
    y_a, sf = pl.pallas_call(
        _scan_body,
        grid=(NC,),
        in_specs=[c_spec, c_spec, t_spec, t_spec, t_spec, s_spec,
                  pl.BlockSpec((NH, HID), lambda c: (0, 0))],
        out_specs=[t_spec, s_spec],
        out_shape=[jax.ShapeDtypeStruct((B, S, HID), f32),
                   jax.ShapeDtypeStruct((B, D, HID), f32)],
        scratch_shapes=[pltpu.VMEM((B, D, HID), f32)],
        compiler_params=pltpu.CompilerParams(
            dimension_semantics=("arbitrary",)),
        name="gdn_scan",
    )(kc_in, qc_in, v_in, a_in, b_in, s0, segt_bf)

    RO = 512
    no = BS // RO
    yf = y_a.reshape(BS, HID).astype(jnp.bfloat16)
    of = pl.pallas_call(
        _out_body,
        grid=(no,),
        in_specs=[pl.BlockSpec((RO, HID), lambda i: (i, 0)),
                  pl.BlockSpec((HID, HID), lambda i: (0, 0))],
        out_specs=pl.BlockSpec((RO, HID), lambda i: (i, 0)),
        out_shape=jax.ShapeDtypeStruct((BS, HID), f32),
        compiler_params=pltpu.CompilerParams(
            dimension_semantics=("arbitrary",)),
        name="gdn_out",
    )(yf, Wo.astype(jnp.bfloat16))

    out = of.reshape(B, S, HID)
    final_state = sf.reshape(B, D, NH, D).transpose(0, 2, 3, 1)
    return out, final_state


# speedup vs baseline: 1.1740x; 1.0430x over previous
import numpy as np

import jax
import jax.numpy as jnp
from jax.experimental import pallas as pl
from jax.experimental.pallas import tpu as pltpu

B, S, HID = 2, 1024, 1024
NH, D = 16, 64
CLIP = 5.0
EPS = 1e-6

CH = 128
NC = S // CH
UNROLL = 16
TL = 128


def _proj_body(x_ref, wq_ref, wk_ref, wv_ref, wab_ref, bab_ref, m_ref,
               seg_ref, segt_ref,
               qc_ref, kc_ref, v_ref, a_ref, b_ref):
    x = x_ref[...]
    seg = seg_ref[...]
    segt = segt_ref[...]
    m = m_ref[...]

    pq = jnp.dot(x, wq_ref[...], preferred_element_type=jnp.float32)
    sq = jnp.dot(pq * pq, seg, preferred_element_type=jnp.float32)
    dq = 1.0 / jnp.maximum(jnp.sqrt(sq), EPS)
    qn = pq * jnp.dot(dq, segt, preferred_element_type=jnp.float32)
    qc_ref[...] = (qn * m).astype(jnp.bfloat16)

    pk = jnp.dot(x, wk_ref[...], preferred_element_type=jnp.float32)
    sk = jnp.dot(pk * pk, seg, preferred_element_type=jnp.float32)
    dk = 1.0 / jnp.maximum(jnp.sqrt(sk), EPS)
    kc_ref[...] = (pk * jnp.dot(dk, segt,
                                preferred_element_type=jnp.float32)
                   ).astype(jnp.bfloat16)

    ab = jnp.dot(x, wab_ref[...], preferred_element_type=jnp.float32)
    ab = jax.nn.sigmoid(ab + bab_ref[...])
    al = jnp.where(m > 0.0, ab[:, :NH], 1.0)
    be = ab[:, NH:] * m
    a_ref[...] = jnp.dot(al, segt, preferred_element_type=jnp.float32)
    b_ref[...] = jnp.dot(be, segt, preferred_element_type=jnp.float32)

    v_ref[...] = jnp.tanh(
        jnp.dot(x, wv_ref[...], preferred_element_type=jnp.float32))


def _sumk(t):
    return jnp.sum(t, axis=0, keepdims=True)


def _scan_body(kc_ref, qc_ref, v_ref, a_ref, b_ref, s0_ref, segt_ref,
               y_ref, sf_ref, st_ref):
    c = pl.program_id(0)

    @pl.when(c == 0)
    def _():
        st_ref[...] = s0_ref[...]

    segt = segt_ref[...]

    dn = (((0,), (0,)), ((), ()))

    def step(t, _):
        for b in range(B):
            kcolt = kc_ref[b, t]
            qcolt = qc_ref[b, t]
            vrow = v_ref[b, pl.ds(t, 1), :]
            arow = a_ref[b, pl.ds(t, 1), :]
            brow = b_ref[b, pl.ds(t, 1), :]
            yparts = []
            for lt in range(HID // TL):
                sl = slice(lt * TL, (lt + 1) * TL)
                hsl = slice(lt * (TL // D), (lt + 1) * (TL // D))
                kbc = jax.lax.dot_general(
                    kcolt[hsl, :], segt[hsl, sl], dn,
                    preferred_element_type=jnp.float32)
                qbc = jax.lax.dot_general(
                    qcolt[hsl, :], segt[hsl, sl], dn,
                    preferred_element_type=jnp.float32)
                st = st_ref[b, :, sl]
                rd = _sumk(st * kbc)
                dsc = (vrow[:, sl] - rd) * brow[:, sl]
                ns = jax.lax.clamp(-CLIP, arow[:, sl] * st + dsc * kbc,
                                   CLIP)
                st_ref[b, :, sl] = ns
                yparts.append(_sumk(ns * qbc))
            y_ref[b, pl.ds(t, 1), :] = jnp.concatenate(yparts, axis=1)
        return ()

    jax.lax.fori_loop(0, CH, step, (), unroll=UNROLL)
    sf_ref[...] = st_ref[...]


def _out_body(y_ref, wo_ref, o_ref):
    o_ref[...] = jnp.dot(y_ref[...], wo_ref[...],
                         preferred_element_type=jnp.float32)


def kernel(x, Wq, Wk, Wv, Wa, ba, Wb, bb, Wo, state0, attention_mask):
    f32 = jnp.float32
    BS = B * S

    cols = np.arange(HID)
    seg = np.zeros((HID, NH), np.float32)
    seg[cols, cols // D] = 1.0
    seg = jnp.asarray(seg)
    segt = seg.T
    segt_bf = segt.astype(jnp.bfloat16)

    xf = x.reshape(BS, HID)
    wab = jnp.concatenate([Wa, Wb], axis=1)
    bab = jnp.concatenate([ba, bb]).reshape(1, 2 * NH)
    mf = attention_mask.astype(f32).reshape(BS, 1)

    RM = 256
    nm = BS // RM
    row_spec = pl.BlockSpec((RM, HID), lambda i: (i, 0))
    full = lambda *shape: pl.BlockSpec(shape, lambda i: (0,) * len(shape))
    qc_a, kc_a, v_a, a_a, b_a = pl.pallas_call(
        _proj_body,
        grid=(nm,),
        in_specs=[row_spec, full(HID, HID), full(HID, HID), full(HID, HID),
                  full(HID, 2 * NH), full(1, 2 * NH),
                  pl.BlockSpec((RM, 1), lambda i: (i, 0)),
                  full(HID, NH), full(NH, HID)],
        out_specs=[row_spec] * 5,
        out_shape=[jax.ShapeDtypeStruct((BS, HID), jnp.bfloat16)] * 2
        + [jax.ShapeDtypeStruct((BS, HID), f32)] * 3,
        compiler_params=pltpu.CompilerParams(
            dimension_semantics=("arbitrary",)),
        name="gdn_proj",
    )(xf, Wq, Wk, Wv, wab, bab, mf, seg, segt)

    kc_in = kc_a.reshape(B, S, NH, D)
    qc_in = qc_a.reshape(B, S, NH, D)
    v_in = v_a.reshape(B, S, HID)
    a_in = a_a.reshape(B, S, HID)
    b_in = b_a.reshape(B, S, HID)
    s0 = state0.transpose(0, 3, 1, 2).reshape(B, D, HID)

    t_spec = pl.BlockSpec((B, CH, HID), lambda c: (0, c, 0))
    c_spec = pl.BlockSpec((B, CH, NH, D), lambda c: (0, c, 0, 0))
    s_spec = pl.BlockSpec((B, D, HID), lambda c: (0, 0, 0))
    y_a, sf = pl.pallas_call(
        _scan_body,
        grid=(NC,),
        in_specs=[c_spec, c_spec, t_spec, t_spec, t_spec, s_spec,
                  pl.BlockSpec((NH, HID), lambda c: (0, 0))],
        out_specs=[t_spec, s_spec],
        out_shape=[jax.ShapeDtypeStruct((B, S, HID), f32),
                   jax.ShapeDtypeStruct((B, D, HID), f32)],
        scratch_shapes=[pltpu.VMEM((B, D, HID), f32)],
        compiler_params=pltpu.CompilerParams(
            dimension_semantics=("arbitrary",)),
        name="gdn_scan",
    )(kc_in, qc_in, v_in, a_in, b_in, s0, segt_bf)

    RO = 512
    no = BS // RO
    yf = y_a.reshape(BS, HID).astype(jnp.bfloat16)
    of = pl.pallas_call(
        _out_body,
        grid=(no,),
        in_specs=[pl.BlockSpec((RO, HID), lambda i: (i, 0)),
                  pl.BlockSpec((HID, HID), lambda i: (0, 0))],
        out_specs=pl.BlockSpec((RO, HID), lambda i: (i, 0)),
        out_shape=jax.ShapeDtypeStruct((BS, HID), f32),
        compiler_params=pltpu.CompilerParams(
            dimension_semantics=("arbitrary",)),
        name="gdn_out",
    )(yf, Wo.astype(jnp.bfloat16))

    out = of.reshape(B, S, HID)
    final_state = sf.reshape(B, D, NH, D).transpose(0, 2, 3, 1)
    return out, final_state


# speedup vs baseline: 1.2066x; 1.0278x over previous
import numpy as np

import jax
import jax.numpy as jnp
from jax.experimental import pallas as pl
from jax.experimental.pallas import tpu as pltpu

B, S, HID = 2, 1024, 1024
NH, D = 16, 64
CLIP = 5.0
EPS = 1e-6

CH = 128
NC = S // CH
UNROLL = 32
TL = 128


def _proj_body(x_ref, wq_ref, wk_ref, wv_ref, wab_ref, bab_ref, m_ref,
               seg_ref, segt_ref,
               qc_ref, kc_ref, v_ref, a_ref, b_ref):
    x = x_ref[...]
    seg = seg_ref[...]
    segt = segt_ref[...]
    m = m_ref[...]

    pq = jnp.dot(x, wq_ref[...], preferred_element_type=jnp.float32)
    sq = jnp.dot(pq * pq, seg, preferred_element_type=jnp.float32)
    dq = 1.0 / jnp.maximum(jnp.sqrt(sq), EPS)
    qn = pq * jnp.dot(dq, segt, preferred_element_type=jnp.float32)
    qc_ref[...] = (qn * m).astype(jnp.bfloat16)

    pk = jnp.dot(x, wk_ref[...], preferred_element_type=jnp.float32)
    sk = jnp.dot(pk * pk, seg, preferred_element_type=jnp.float32)
    dk = 1.0 / jnp.maximum(jnp.sqrt(sk), EPS)
    kc_ref[...] = (pk * jnp.dot(dk, segt,
                                preferred_element_type=jnp.float32)
                   ).astype(jnp.bfloat16)

    ab = jnp.dot(x, wab_ref[...], preferred_element_type=jnp.float32)
    ab = jax.nn.sigmoid(ab + bab_ref[...])
    al = jnp.where(m > 0.0, ab[:, :NH], 1.0)
    be = ab[:, NH:] * m
    a_ref[...] = jnp.dot(al, segt, preferred_element_type=jnp.float32)
    b_ref[...] = jnp.dot(be, segt, preferred_element_type=jnp.float32)

    v_ref[...] = jnp.tanh(
        jnp.dot(x, wv_ref[...], preferred_element_type=jnp.float32))


def _sumk(t):
    return jnp.sum(t, axis=0, keepdims=True)


def _scan_body(kc_ref, qc_ref, v_ref, a_ref, b_ref, s0_ref, segt_ref,
               y_ref, sf_ref, st_ref):
    c = pl.program_id(0)

    @pl.when(c == 0)
    def _():
        st_ref[...] = s0_ref[...]

    segt = segt_ref[...]

    dn = (((0,), (0,)), ((), ()))

    def step(t, _):
        for b in range(B):
            kcolt = kc_ref[b, t]
            qcolt = qc_ref[b, t]
            vrow = v_ref[b, pl.ds(t, 1), :]
            arow = a_ref[b, pl.ds(t, 1), :]
            brow = b_ref[b, pl.ds(t, 1), :]
            yparts = []
            for lt in range(HID // TL):
                sl = slice(lt * TL, (lt + 1) * TL)
                hsl = slice(lt * (TL // D), (lt + 1) * (TL // D))
                kbc = jax.lax.dot_general(
                    kcolt[hsl, :], segt[hsl, sl], dn,
                    preferred_element_type=jnp.float32)
                qbc = jax.lax.dot_general(
                    qcolt[hsl, :], segt[hsl, sl], dn,
                    preferred_element_type=jnp.float32)
                st = st_ref[b, :, sl]
                rd = _sumk(st * kbc)
                dsc = (vrow[:, sl] - rd) * brow[:, sl]
                ns = jax.lax.clamp(-CLIP, arow[:, sl] * st + dsc * kbc,
                                   CLIP)
                st_ref[b, :, sl] = ns
                yparts.append(_sumk(ns * qbc))
            y_ref[b, pl.ds(t, 1), :] = jnp.concatenate(yparts, axis=1)
        return ()

    jax.lax.fori_loop(0, CH, step, (), unroll=UNROLL)
    sf_ref[...] = st_ref[...]


def _out_body(y_ref, wo_ref, o_ref):
    o_ref[...] = jnp.dot(y_ref[...], wo_ref[...],
                         preferred_element_type=jnp.float32)


def kernel(x, Wq, Wk, Wv, Wa, ba, Wb, bb, Wo, state0, attention_mask):
    f32 = jnp.float32
    BS = B * S

    cols = np.arange(HID)
    seg = np.zeros((HID, NH), np.float32)
    seg[cols, cols // D] = 1.0
    seg = jnp.asarray(seg)
    segt = seg.T
    segt_bf = segt.astype(jnp.bfloat16)

    xf = x.reshape(BS, HID)
    wab = jnp.concatenate([Wa, Wb], axis=1)
    bab = jnp.concatenate([ba, bb]).reshape(1, 2 * NH)
    mf = attention_mask.astype(f32).reshape(BS, 1)

    RM = 256
    nm = BS // RM
    row_spec = pl.BlockSpec((RM, HID), lambda i: (i, 0))
    full = lambda *shape: pl.BlockSpec(shape, lambda i: (0,) * len(shape))
    qc_a, kc_a, v_a, a_a, b_a = pl.pallas_call(
        _proj_body,
        grid=(nm,),
        in_specs=[row_spec, full(HID, HID), full(HID, HID), full(HID, HID),
                  full(HID, 2 * NH), full(1, 2 * NH),
                  pl.BlockSpec((RM, 1), lambda i: (i, 0)),
                  full(HID, NH), full(NH, HID)],
        out_specs=[row_spec] * 5,
        out_shape=[jax.ShapeDtypeStruct((BS, HID), jnp.bfloat16)] * 2
        + [jax.ShapeDtypeStruct((BS, HID), f32)] * 3,
        compiler_params=pltpu.CompilerParams(
            dimension_semantics=("arbitrary",)),
        name="gdn_proj",
    )(xf, Wq, Wk, Wv, wab, bab, mf, seg, segt)

    kc_in = kc_a.reshape(B, S, NH, D)
    qc_in = qc_a.reshape(B, S, NH, D)
    v_in = v_a.reshape(B, S, HID)
    a_in = a_a.reshape(B, S, HID)
    b_in = b_a.reshape(B, S, HID)
    s0 = state0.transpose(0, 3, 1, 2).reshape(B, D, HID)

    t_spec = pl.BlockSpec((B, CH, HID), lambda c: (0, c, 0))
    c_spec = pl.BlockSpec((B, CH, NH, D), lambda c: (0, c, 0, 0))
    s_spec = pl.BlockSpec((B, D, HID), lambda c: (0, 0, 0))
    y_a, sf = pl.pallas_call(
        _scan_body,
        grid=(NC,),
        in_specs=[c_spec, c_spec, t_spec, t_spec, t_spec, s_spec,
                  pl.BlockSpec((NH, HID), lambda c: (0, 0))],
        out_specs=[t_spec, s_spec],
        out_shape=[jax.ShapeDtypeStruct((B, S, HID), f32),
                   jax.ShapeDtypeStruct((B, D, HID), f32)],
        scratch_shapes=[pltpu.VMEM((B, D, HID), f32)],
        compiler_params=pltpu.CompilerParams(
            dimension_semantics=("arbitrary",)),
        name="gdn_scan",
    )(kc_in, qc_in, v_in, a_in, b_in, s0, segt_bf)

    RO = 512
    no = BS // RO
    yf = y_a.reshape(BS, HID).astype(jnp.bfloat16)
    of = pl.pallas_call(
        _out_body,
        grid=(no,),
        in_specs=[pl.BlockSpec((RO, HID), lambda i: (i, 0)),
                  pl.BlockSpec((HID, HID), lambda i: (0, 0))],
        out_specs=pl.BlockSpec((RO, HID), lambda i: (i, 0)),
        out_shape=jax.ShapeDtypeStruct((BS, HID), f32),
        compiler_params=pltpu.CompilerParams(
            dimension_semantics=("arbitrary",)),
        name="gdn_out",
    )(yf, Wo.astype(jnp.bfloat16))

    out = of.reshape(B, S, HID)
    final_state = sf.reshape(B, D, NH, D).transpose(0, 2, 3, 1)
    return out, final_state
